# baseline (speedup 1.0000x reference)
import jax
import jax.numpy as jnp
from jax.experimental import pallas as pl

BATCH = 16384
E = 64
BT = 512


def _body(x_ref, c_ref, r_ref, out_ref, idx_ref):
    x = x_ref[...]
    c = c_ref[...]
    r = r_ref[...]

    cn2 = jnp.sum(c * c, axis=1)
    xn2 = jnp.sum(x * x, axis=1)
    g = jax.lax.dot_general(x, c, (((1,), (1,)), ((), ())),
                            precision=jax.lax.Precision.HIGHEST,
                            preferred_element_type=jnp.float32)
    d2m = xn2[:, None] + (cn2[None, :] - 2.0 * g)

    lane = jax.lax.broadcasted_iota(jnp.int32, (BT, E), 1)
    a1 = jnp.argmin(d2m, axis=1)
    masked = jnp.where(lane == a1[:, None], jnp.inf, d2m)
    a2 = jnp.argmin(masked, axis=1)

    h1 = (lane == a1[:, None]).astype(jnp.float32)
    h2 = (lane == a2[:, None]).astype(jnp.float32)
    c1 = jnp.dot(h1, c, precision=jax.lax.Precision.HIGHEST,
                 preferred_element_type=jnp.float32)
    c2 = jnp.dot(h2, c, precision=jax.lax.Precision.HIGHEST,
                 preferred_element_type=jnp.float32)

    dx1 = x - c1
    dx2 = x - c2
    s1 = jnp.sqrt(jnp.sum(dx1 * dx1, axis=1))
    s2 = jnp.sqrt(jnp.sum(dx2 * dx2, axis=1))

    pred = (s2 < s1) | ((s2 == s1) & (a2 < a1))
    best = jnp.where(pred, a2, a1)
    mind = jnp.where(pred, s2, s1)
    csel = jnp.where(pred[:, None], c2, c1)
    rsel = jnp.where(pred, jnp.sum(h2 * r, axis=1), jnp.sum(h1 * r, axis=1))

    s = 0.1 * jnp.exp(-mind / (rsel + 1e-8))
    out_ref[...] = x * (1.0 - s)[:, None] + csel * s[:, None]
    idx_ref[...] = best[:, None].astype(jnp.int32)


def kernel(expert_activations, attractor_centers, attraction_radii):
    radii2d = attraction_radii.reshape(1, E)
    attracted, closest = pl.pallas_call(
        _body,
        grid=(BATCH // BT,),
        in_specs=[
            pl.BlockSpec((BT, E), lambda i: (i, 0)),
            pl.BlockSpec((E, E), lambda i: (0, 0)),
            pl.BlockSpec((1, E), lambda i: (0, 0)),
        ],
        out_specs=[
            pl.BlockSpec((BT, E), lambda i: (i, 0)),
            pl.BlockSpec((BT, 1), lambda i: (i, 0)),
        ],
        out_shape=[
            jax.ShapeDtypeStruct((BATCH, E), jnp.float32),
            jax.ShapeDtypeStruct((BATCH, 1), jnp.int32),
        ],
    )(expert_activations, attractor_centers, radii2d)
    return attracted, closest.reshape(BATCH)


# speedup vs baseline: 4.5704x; 4.5704x over previous
import jax
import jax.numpy as jnp
from jax.experimental import pallas as pl

BATCH = 16384
E = 64
BT = 512
HI = jax.lax.Precision.HIGHEST


def _body(x_ref, c_ref, r_ref, out_ref, idx_ref):
    x = x_ref[...]
    c = c_ref[...]
    r = r_ref[...]

    eye = jnp.eye(E, dtype=jnp.float32)
    xt = jax.lax.dot_general(eye, x, (((1,), (1,)), ((), ())),
                             precision=HI,
                             preferred_element_type=jnp.float32)

    cn2 = jnp.sum(c * c, axis=1, keepdims=True)
    g = jax.lax.dot_general(c, x, (((1,), (1,)), ((), ())),
                            precision=HI,
                            preferred_element_type=jnp.float32)
    s_rank = 0.5 * cn2 - g

    sub = jax.lax.broadcasted_iota(jnp.int32, (E, BT), 0)
    a1 = jnp.argmin(s_rank, axis=0)
    masked = jnp.where(sub == a1[None, :], jnp.inf, s_rank)
    a2 = jnp.argmin(masked, axis=0)

    h1 = (sub == a1[None, :]).astype(jnp.float32)
    h2 = (sub == a2[None, :]).astype(jnp.float32)
    c1t = jax.lax.dot_general(c, h1, (((0,), (0,)), ((), ())),
                              precision=HI,
                              preferred_element_type=jnp.float32)
    c2t = jax.lax.dot_general(c, h2, (((0,), (0,)), ((), ())),
                              precision=HI,
                              preferred_element_type=jnp.float32)

    dx1 = xt - c1t
    dx2 = xt - c2t
    s1 = jnp.sqrt(jnp.sum(dx1 * dx1, axis=0))
    s2 = jnp.sqrt(jnp.sum(dx2 * dx2, axis=0))

    pred = (s2 < s1) | ((s2 == s1) & (a2 < a1))
    best = jnp.where(pred, a2, a1)
    mind = jnp.where(pred, s2, s1)
    cselt = jnp.where(pred[None, :], c2t, c1t)
    rsel = jnp.sum(jnp.where(pred[None, :], h2, h1) * r, axis=0)

    s = 0.1 * jnp.exp(-mind / (rsel + 1e-8))
    outt = xt * (1.0 - s)[None, :] + cselt * s[None, :]
    out_ref[...] = jax.lax.dot_general(outt, eye, (((0,), (0,)), ((), ())),
                                       precision=HI,
                                       preferred_element_type=jnp.float32)
    idx_ref[...] = best[None, None, :].astype(jnp.int32)


def kernel(expert_activations, attractor_centers, attraction_radii):
    radii_col = attraction_radii.reshape(E, 1)
    attracted, closest = pl.pallas_call(
        _body,
        grid=(BATCH // BT,),
        in_specs=[
            pl.BlockSpec((BT, E), lambda i: (i, 0)),
            pl.BlockSpec((E, E), lambda i: (0, 0)),
            pl.BlockSpec((E, 1), lambda i: (0, 0)),
        ],
        out_specs=[
            pl.BlockSpec((BT, E), lambda i: (i, 0)),
            pl.BlockSpec((1, 1, BT), lambda i: (i, 0, 0)),
        ],
        out_shape=[
            jax.ShapeDtypeStruct((BATCH, E), jnp.float32),
            jax.ShapeDtypeStruct((BATCH // BT, 1, BT), jnp.int32),
        ],
    )(expert_activations, attractor_centers, radii_col)
    return attracted, closest.reshape(BATCH)


# speedup vs baseline: 5.6193x; 1.2295x over previous
import jax
import jax.numpy as jnp
from jax.experimental import pallas as pl

BATCH = 16384
E = 64
BT = 2048
HI = jax.lax.Precision.HIGHEST


def _body(x_ref, c_ref, r_ref, out_ref, idx_ref):
    x = x_ref[...]
    c = c_ref[...]
    r = r_ref[...]

    eye = jnp.eye(E, dtype=jnp.float32)
    xt = jax.lax.dot_general(eye, x, (((1,), (1,)), ((), ())),
                             precision=HI,
                             preferred_element_type=jnp.float32)

    cn2 = jnp.sum(c * c, axis=1, keepdims=True)
    g = jax.lax.dot_general(c, x, (((1,), (1,)), ((), ())),
                            precision=HI,
                            preferred_element_type=jnp.float32)
    s_rank = 0.5 * cn2 - g

    sub = jax.lax.broadcasted_iota(jnp.int32, (E, BT), 0)
    a1 = jnp.argmin(s_rank, axis=0)
    masked = jnp.where(sub == a1[None, :], jnp.inf, s_rank)
    a2 = jnp.argmin(masked, axis=0)

    h1 = (sub == a1[None, :]).astype(jnp.float32)
    h2 = (sub == a2[None, :]).astype(jnp.float32)
    c1t = jax.lax.dot_general(c, h1, (((0,), (0,)), ((), ())),
                              precision=HI,
                              preferred_element_type=jnp.float32)
    c2t = jax.lax.dot_general(c, h2, (((0,), (0,)), ((), ())),
                              precision=HI,
                              preferred_element_type=jnp.float32)

    dx1 = xt - c1t
    dx2 = xt - c2t
    s1 = jnp.sqrt(jnp.sum(dx1 * dx1, axis=0))
    s2 = jnp.sqrt(jnp.sum(dx2 * dx2, axis=0))

    pred = (s2 < s1) | ((s2 == s1) & (a2 < a1))
    best = jnp.where(pred, a2, a1)
    mind = jnp.where(pred, s2, s1)
    cselt = jnp.where(pred[None, :], c2t, c1t)
    rsel = jnp.sum(jnp.where(pred[None, :], h2, h1) * r, axis=0)

    s = 0.1 * jnp.exp(-mind / (rsel + 1e-8))
    outt = xt * (1.0 - s)[None, :] + cselt * s[None, :]
    out_ref[...] = jax.lax.dot_general(outt, eye, (((0,), (0,)), ((), ())),
                                       precision=HI,
                                       preferred_element_type=jnp.float32)
    idx_ref[...] = best[None, None, :].astype(jnp.int32)


def kernel(expert_activations, attractor_centers, attraction_radii):
    radii_col = attraction_radii.reshape(E, 1)
    attracted, closest = pl.pallas_call(
        _body,
        grid=(BATCH // BT,),
        in_specs=[
            pl.BlockSpec((BT, E), lambda i: (i, 0)),
            pl.BlockSpec((E, E), lambda i: (0, 0)),
            pl.BlockSpec((E, 1), lambda i: (0, 0)),
        ],
        out_specs=[
            pl.BlockSpec((BT, E), lambda i: (i, 0)),
            pl.BlockSpec((1, 1, BT), lambda i: (i, 0, 0)),
        ],
        out_shape=[
            jax.ShapeDtypeStruct((BATCH, E), jnp.float32),
            jax.ShapeDtypeStruct((BATCH // BT, 1, BT), jnp.int32),
        ],
    )(expert_activations, attractor_centers, radii_col)
    return attracted, closest.reshape(BATCH)


# speedup vs baseline: 8.0884x; 1.4394x over previous
import jax
import jax.numpy as jnp
from jax.experimental import pallas as pl

BATCH = 16384
E = 64
BT = 2048
HI = jax.lax.Precision.HIGHEST


def _body(x_ref, c_ref, r_ref, out_ref, idx_ref):
    x = x_ref[...]
    c = c_ref[...]
    r = r_ref[...]

    xt = jnp.swapaxes(x, 0, 1)

    cn2 = jnp.sum(c * c, axis=1, keepdims=True)
    g = jax.lax.dot_general(c, x, (((1,), (1,)), ((), ())),
                            precision=HI,
                            preferred_element_type=jnp.float32)
    s_rank = 0.5 * cn2 - g

    sub = jax.lax.broadcasted_iota(jnp.int32, (E, BT), 0)
    a1 = jnp.argmin(s_rank, axis=0)
    masked = jnp.where(sub == a1[None, :], jnp.inf, s_rank)
    a2 = jnp.argmin(masked, axis=0)

    h1 = (sub == a1[None, :]).astype(jnp.float32)
    h2 = (sub == a2[None, :]).astype(jnp.float32)
    c1t = jax.lax.dot_general(c, h1, (((0,), (0,)), ((), ())),
                              precision=HI,
                              preferred_element_type=jnp.float32)
    c2t = jax.lax.dot_general(c, h2, (((0,), (0,)), ((), ())),
                              precision=HI,
                              preferred_element_type=jnp.float32)

    dx1 = xt - c1t
    dx2 = xt - c2t
    s1 = jnp.sqrt(jnp.sum(dx1 * dx1, axis=0))
    s2 = jnp.sqrt(jnp.sum(dx2 * dx2, axis=0))

    pred = (s2 < s1) | ((s2 == s1) & (a2 < a1))
    best = jnp.where(pred, a2, a1)
    mind = jnp.where(pred, s2, s1)
    cselt = jnp.where(pred[None, :], c2t, c1t)
    rt = jnp.swapaxes(r, 0, 1)
    r1 = jax.lax.dot_general(rt, h1, (((1,), (0,)), ((), ())),
                             precision=HI,
                             preferred_element_type=jnp.float32)
    r2 = jax.lax.dot_general(rt, h2, (((1,), (0,)), ((), ())),
                             precision=HI,
                             preferred_element_type=jnp.float32)
    rsel = jnp.where(pred, r2[0], r1[0])

    s = 0.1 * jnp.exp(-mind / (rsel + 1e-8))
    outt = xt * (1.0 - s)[None, :] + cselt * s[None, :]
    out_ref[...] = jnp.swapaxes(outt, 0, 1)
    idx_ref[...] = best[None, None, :].astype(jnp.int32)


def kernel(expert_activations, attractor_centers, attraction_radii):
    radii_col = attraction_radii.reshape(E, 1)
    attracted, closest = pl.pallas_call(
        _body,
        grid=(BATCH // BT,),
        in_specs=[
            pl.BlockSpec((BT, E), lambda i: (i, 0)),
            pl.BlockSpec((E, E), lambda i: (0, 0)),
            pl.BlockSpec((E, 1), lambda i: (0, 0)),
        ],
        out_specs=[
            pl.BlockSpec((BT, E), lambda i: (i, 0)),
            pl.BlockSpec((1, 1, BT), lambda i: (i, 0, 0)),
        ],
        out_shape=[
            jax.ShapeDtypeStruct((BATCH, E), jnp.float32),
            jax.ShapeDtypeStruct((BATCH // BT, 1, BT), jnp.int32),
        ],
    )(expert_activations, attractor_centers, radii_col)
    return attracted, closest.reshape(BATCH)


# speedup vs baseline: 8.1382x; 1.0062x over previous
import jax
import jax.numpy as jnp
from jax.experimental import pallas as pl

BATCH = 16384
E = 64
BT = 4096
HI = jax.lax.Precision.HIGHEST


def _body(x_ref, c_ref, r_ref, out_ref, idx_ref):
    x = x_ref[...]
    c = c_ref[...]
    r = r_ref[...]

    xt = jnp.swapaxes(x, 0, 1)

    cn2 = jnp.sum(c * c, axis=1, keepdims=True)
    g = jax.lax.dot_general(c, x, (((1,), (1,)), ((), ())),
                            precision=HI,
                            preferred_element_type=jnp.float32)
    s_rank = 0.5 * cn2 - g

    sub = jax.lax.broadcasted_iota(jnp.int32, (E, BT), 0)
    a1 = jnp.argmin(s_rank, axis=0)
    masked = jnp.where(sub == a1[None, :], jnp.inf, s_rank)
    a2 = jnp.argmin(masked, axis=0)

    h1 = (sub == a1[None, :]).astype(jnp.float32)
    h2 = (sub == a2[None, :]).astype(jnp.float32)
    c1t = jax.lax.dot_general(c, h1, (((0,), (0,)), ((), ())),
                              precision=HI,
                              preferred_element_type=jnp.float32)
    c2t = jax.lax.dot_general(c, h2, (((0,), (0,)), ((), ())),
                              precision=HI,
                              preferred_element_type=jnp.float32)

    dx1 = xt - c1t
    dx2 = xt - c2t
    s1 = jnp.sqrt(jnp.sum(dx1 * dx1, axis=0))
    s2 = jnp.sqrt(jnp.sum(dx2 * dx2, axis=0))

    pred = (s2 < s1) | ((s2 == s1) & (a2 < a1))
    best = jnp.where(pred, a2, a1)
    mind = jnp.where(pred, s2, s1)
    cselt = jnp.where(pred[None, :], c2t, c1t)
    rt = jnp.swapaxes(r, 0, 1)
    r1 = jax.lax.dot_general(rt, h1, (((1,), (0,)), ((), ())),
                             precision=HI,
                             preferred_element_type=jnp.float32)
    r2 = jax.lax.dot_general(rt, h2, (((1,), (0,)), ((), ())),
                             precision=HI,
                             preferred_element_type=jnp.float32)
    rsel = jnp.where(pred, r2[0], r1[0])

    s = 0.1 * jnp.exp(-mind / (rsel + 1e-8))
    outt = xt * (1.0 - s)[None, :] + cselt * s[None, :]
    out_ref[...] = jnp.swapaxes(outt, 0, 1)
    idx_ref[...] = best[None, None, :].astype(jnp.int32)


def kernel(expert_activations, attractor_centers, attraction_radii):
    radii_col = attraction_radii.reshape(E, 1)
    attracted, closest = pl.pallas_call(
        _body,
        grid=(BATCH // BT,),
        in_specs=[
            pl.BlockSpec((BT, E), lambda i: (i, 0)),
            pl.BlockSpec((E, E), lambda i: (0, 0)),
            pl.BlockSpec((E, 1), lambda i: (0, 0)),
        ],
        out_specs=[
            pl.BlockSpec((BT, E), lambda i: (i, 0)),
            pl.BlockSpec((1, 1, BT), lambda i: (i, 0, 0)),
        ],
        out_shape=[
            jax.ShapeDtypeStruct((BATCH, E), jnp.float32),
            jax.ShapeDtypeStruct((BATCH // BT, 1, BT), jnp.int32),
        ],
    )(expert_activations, attractor_centers, radii_col)
    return attracted, closest.reshape(BATCH)
